# baseline (speedup 1.0000x reference)
import functools

import jax
import jax.numpy as jnp
from jax.experimental import pallas as pl
from jax.experimental.pallas import tpu as pltpu

_CACHE_BLKS = 1


def _fused_kernel(g_ref, x_ref, w_ref, out_ref, s32_ref, s16_ref, norm_ref,
                  cache_ref, *, br, half, nblk):
    i = pl.program_id(0)
    n = g_ref.shape[1]
    lo = (i % nblk) * br

    @pl.when(i < nblk)
    def _():
        ones = jnp.ones((n, 128), dtype=jnp.float32)
        rs = jnp.dot(g_ref[...], ones, preferred_element_type=jnp.float32,
                     precision=jax.lax.Precision.DEFAULT)[:, :1]
        nrm = jax.lax.rsqrt(jnp.maximum(rs, 1.0))
        norm_ref[pl.ds(lo, br), :] = nrm
        x = x_ref[...]
        h1 = jnp.dot(x, w_ref[0], preferred_element_type=jnp.float32,
                     precision=jax.lax.Precision.HIGHEST)
        h2 = jnp.dot(x, w_ref[1], preferred_element_type=jnp.float32,
                     precision=jax.lax.Precision.HIGHEST)
        rows = lo + jax.lax.broadcasted_iota(jnp.int32, (br, 1), 0)
        h = jnp.where(rows < half, h1, h2)
        s = h * nrm
        s32_ref[pl.ds(lo, br), :] = s
        s16_ref[pl.ds(lo, br), :] = s.astype(jnp.bfloat16)

        @pl.when(i >= nblk - _CACHE_BLKS)
        def _():
            clo = (i - (nblk - _CACHE_BLKS)) * br
            def fill(c, _):
                cache_ref[pl.ds(clo + c * 8, 8), :] = (
                    g_ref[pl.ds(c * 8, 8), :].astype(jnp.bfloat16))
                return 0
            jax.lax.fori_loop(0, br // 8, fill, 0)

    @pl.when(i >= nblk)
    def _():
        nrm = norm_ref[pl.ds(lo, br), :]

        @pl.when(i < 2 * nblk - _CACHE_BLKS)
        def _():
            acc = jnp.dot(g_ref[...], s32_ref[...],
                          preferred_element_type=jnp.float32,
                          precision=jax.lax.Precision.DEFAULT)
            out_ref[...] = acc * nrm

        @pl.when(i >= 2 * nblk - _CACHE_BLKS)
        def _():
            clo = jnp.maximum(i - (2 * nblk - _CACHE_BLKS), 0) * br
            g = cache_ref[pl.ds(clo, br), :]
            acc = jnp.dot(g, s16_ref[...], preferred_element_type=jnp.float32)
            out_ref[...] = acc * nrm


def kernel(graph, drug_f, disease_f, drug_w, disease_w):
    n = graph.shape[0]
    half = drug_f.shape[0]
    d = drug_f.shape[1]
    br = 400 if n % 400 == 0 else n
    nblk = n // br

    x = jnp.concatenate([drug_f, disease_f], axis=0)
    w = jnp.stack([drug_w, disease_w], axis=0)

    def g_index(i):
        j = jnp.where(i < nblk, i, i - nblk)
        return (jnp.minimum(j, nblk - 1 - _CACHE_BLKS * (i // nblk)), 0)

    out = pl.pallas_call(
        functools.partial(_fused_kernel, br=br, half=half, nblk=nblk),
        grid=(2 * nblk,),
        in_specs=[
            pl.BlockSpec((br, n), g_index),
            pl.BlockSpec((br, d), lambda i: (i % nblk, 0)),
            pl.BlockSpec((2, d, d), lambda i: (0, 0, 0)),
        ],
        out_specs=pl.BlockSpec(
            (br, d), lambda i: (jnp.maximum(i - nblk, 0), 0)),
        out_shape=jax.ShapeDtypeStruct((n, d), jnp.float32),
        scratch_shapes=[
            pltpu.VMEM((n, d), jnp.float32),
            pltpu.VMEM((n, d), jnp.bfloat16),
            pltpu.VMEM((n, 1), jnp.float32),
            pltpu.VMEM((_CACHE_BLKS * br, n), jnp.bfloat16),
        ],
        compiler_params=pltpu.CompilerParams(
            dimension_semantics=("arbitrary",)),
    )(graph, x, w)
    return out


# speedup vs baseline: 1.0118x; 1.0118x over previous
import functools

import jax
import jax.numpy as jnp
from jax.experimental import pallas as pl
from jax.experimental.pallas import tpu as pltpu

_CACHE_BLKS = 2


def _fused_kernel(g_ref, x_ref, w_ref, out_ref, s16_ref, norm_ref,
                  cache_ref, *, br, half, nblk):
    i = pl.program_id(0)
    n = g_ref.shape[1]
    lo = pl.multiple_of((i % nblk) * br, br)

    @pl.when(i < nblk)
    def _():
        ones = jnp.ones((n, 128), dtype=jnp.float32)
        rs = jnp.dot(g_ref[...], ones, preferred_element_type=jnp.float32,
                     precision=jax.lax.Precision.DEFAULT)[:, :1]
        nrm = jax.lax.rsqrt(jnp.maximum(rs, 1.0))
        norm_ref[pl.ds(lo, br), :] = nrm
        x = x_ref[...]
        h1 = jnp.dot(x, w_ref[0], preferred_element_type=jnp.float32,
                     precision=jax.lax.Precision.HIGHEST)
        h2 = jnp.dot(x, w_ref[1], preferred_element_type=jnp.float32,
                     precision=jax.lax.Precision.HIGHEST)
        rows = lo + jax.lax.broadcasted_iota(jnp.int32, (br, 1), 0)
        h = jnp.where(rows < half, h1, h2)
        s16_ref[pl.ds(lo, br), :] = (h * nrm).astype(jnp.bfloat16)

        @pl.when(i >= nblk - _CACHE_BLKS)
        def _():
            clo = (i - (nblk - _CACHE_BLKS)) * br
            def fill(c, _):
                cache_ref[pl.ds(clo + c * 8, 8), :] = (
                    g_ref[pl.ds(c * 8, 8), :].astype(jnp.bfloat16))
                return 0
            jax.lax.fori_loop(0, br // 8, fill, 0)

    @pl.when(i >= nblk)
    def _():
        nrm = norm_ref[pl.ds(lo, br), :]

        @pl.when(i < 2 * nblk - _CACHE_BLKS)
        def _():
            acc = jnp.dot(g_ref[...], s16_ref[...].astype(jnp.float32),
                          preferred_element_type=jnp.float32,
                          precision=jax.lax.Precision.DEFAULT)
            out_ref[...] = acc * nrm

        @pl.when(i >= 2 * nblk - _CACHE_BLKS)
        def _():
            clo = jnp.maximum(i - (2 * nblk - _CACHE_BLKS), 0) * br
            g = cache_ref[pl.ds(clo, br), :]
            acc = jnp.dot(g, s16_ref[...], preferred_element_type=jnp.float32)
            out_ref[...] = acc * nrm


def kernel(graph, drug_f, disease_f, drug_w, disease_w):
    n = graph.shape[0]
    half = drug_f.shape[0]
    d = drug_f.shape[1]
    br = 400 if n % 400 == 0 else n
    nblk = n // br

    x = jnp.concatenate([drug_f, disease_f], axis=0)
    w = jnp.stack([drug_w, disease_w], axis=0)

    def g_index(i):
        j = jnp.where(i < nblk, i, i - nblk)
        return (jnp.minimum(j, nblk - 1 - _CACHE_BLKS * (i // nblk)), 0)

    out = pl.pallas_call(
        functools.partial(_fused_kernel, br=br, half=half, nblk=nblk),
        grid=(2 * nblk,),
        in_specs=[
            pl.BlockSpec((br, n), g_index),
            pl.BlockSpec((br, d), lambda i: (i % nblk, 0)),
            pl.BlockSpec((2, d, d), lambda i: (0, 0, 0)),
        ],
        out_specs=pl.BlockSpec(
            (br, d), lambda i: (jnp.maximum(i - nblk, 0), 0)),
        out_shape=jax.ShapeDtypeStruct((n, d), jnp.float32),
        scratch_shapes=[
            pltpu.VMEM((n, d), jnp.bfloat16),
            pltpu.VMEM((n, 1), jnp.float32),
            pltpu.VMEM((_CACHE_BLKS * br, n), jnp.bfloat16),
        ],
        compiler_params=pltpu.CompilerParams(
            dimension_semantics=("arbitrary",)),
    )(graph, x, w)
    return out
